# baseline (speedup 1.0000x reference)
import functools

import jax
import jax.numpy as jnp
from jax import lax
from jax.experimental import pallas as pl
from jax.experimental.pallas import tpu as pltpu
from jax.experimental.pallas import tpu_sc as plsc

BATCH = 16384
EMB = 32
NC = 2
NS = 16
NW = NC * NS
BPW = BATCH // NW
LANES = 16
GROUPS = BPW // LANES

_mesh = plsc.VectorSubcoreMesh(core_axis_name="c", subcore_axis_name="s")


@functools.partial(
    pl.kernel,
    mesh=_mesh,
    out_type=jax.ShapeDtypeStruct((BATCH,), jnp.float32),
    scratch_types=[
        pltpu.VMEM((BPW,), jnp.int32),
        pltpu.VMEM((BPW,), jnp.int32),
        pltpu.VMEM((BPW,), jnp.int32),
        pltpu.VMEM((EMB * BPW,), jnp.float32),
        pltpu.VMEM((EMB * BPW,), jnp.float32),
        pltpu.VMEM((EMB * BPW,), jnp.float32),
        pltpu.VMEM((BPW,), jnp.float32),
        pltpu.SemaphoreType.DMA,
    ],
    compiler_params=pltpu.CompilerParams(
        needs_layout_passes=False, use_tc_tiling_on_sc=False
    ),
)
def _transe_sc(h_hbm, r_hbm, t_hbm, ent_hbm, rel_hbm, out_hbm,
               hi, ri, ti, hv, rv, tv, ov, sem):
    wid = lax.axis_index("s") * NC + lax.axis_index("c")
    base = wid * BPW

    pltpu.sync_copy(h_hbm.at[pl.ds(base, BPW)], hi)
    pltpu.sync_copy(r_hbm.at[pl.ds(base, BPW)], ri)
    pltpu.sync_copy(t_hbm.at[pl.ds(base, BPW)], ti)

    copies = []
    for d in range(EMB):
        dst = pl.ds(d * BPW, BPW)
        copies.append(
            pltpu.async_copy(ent_hbm.at[d].at[hi], hv.at[dst], sem)
        )
        copies.append(
            pltpu.async_copy(rel_hbm.at[d].at[ri], rv.at[dst], sem)
        )
        copies.append(
            pltpu.async_copy(ent_hbm.at[d].at[ti], tv.at[dst], sem)
        )
    for cp in copies:
        cp.wait()

    def group_body(g, carry):
        i0 = g * LANES
        acc = jnp.zeros((LANES,), jnp.float32)
        for d in range(EMB):
            src = pl.ds(d * BPW + i0, LANES)
            acc = acc + jnp.abs(hv[src] + rv[src] - tv[src])
        ov[pl.ds(i0, LANES)] = acc
        return carry

    lax.fori_loop(0, GROUPS, group_body, 0)

    pltpu.sync_copy(ov, out_hbm.at[pl.ds(base, BPW)])


def kernel(batch_h, batch_r, batch_t, entity_embds, rel_embds):
    return _transe_sc(batch_h, batch_r, batch_t, entity_embds.T, rel_embds.T)
